# initial kernel scaffold (unmeasured)
import jax
import jax.numpy as jnp
from jax import lax
from jax.experimental import pallas as pl
from jax.experimental.pallas import tpu as pltpu


def kernel(
    x,
):
    def body(*refs):
        pass

    out_shape = jax.ShapeDtypeStruct(..., jnp.float32)
    return pl.pallas_call(body, out_shape=out_shape)(...)


# baseline (device time: 234438 ns/iter reference)
import jax
import jax.numpy as jnp
from jax import lax
from jax.experimental import pallas as pl
from jax.experimental.pallas import tpu as pltpu

N_DEV = 16


def kernel(x):
    m, n = x.shape
    chunk = m // N_DEV

    def body(x_ref, out_ref, rs_buf, rs_send_sems, rs_recv_sems,
             ag_send_sems, ag_recv_sems):
        my = lax.axis_index("i")
        left = lax.rem(my + N_DEV - 1, N_DEV)
        right = lax.rem(my + 1, N_DEV)

        barrier_sem = pltpu.get_barrier_semaphore()
        for nbr in (left, right):
            pl.semaphore_signal(
                barrier_sem, inc=1,
                device_id=(nbr,), device_id_type=pl.DeviceIdType.MESH,
            )
        pl.semaphore_wait(barrier_sem, 2)

        rs_buf[0, :, :] = x_ref[pl.ds(my * chunk, chunk), :]
        for h in range(N_DEV - 1):
            rdma = pltpu.make_async_remote_copy(
                src_ref=rs_buf.at[h],
                dst_ref=rs_buf.at[h + 1],
                send_sem=rs_send_sems.at[h],
                recv_sem=rs_recv_sems.at[h + 1],
                device_id=(right,),
                device_id_type=pl.DeviceIdType.MESH,
            )
            rdma.start()
            rdma.wait()
            recv_idx = lax.rem(my + N_DEV - h - 1, N_DEV)
            rs_buf[h + 1, :, :] = (
                rs_buf[h + 1, :, :] + x_ref[pl.ds(recv_idx * chunk, chunk), :]
            )

        c0 = lax.rem(my + 1, N_DEV)
        out_ref[pl.ds(c0 * chunk, chunk), :] = rs_buf[N_DEV - 1, :, :]

        for t in range(N_DEV - 1):
            s_idx = lax.rem(my + 1 - t + N_DEV, N_DEV)
            rdma = pltpu.make_async_remote_copy(
                src_ref=out_ref.at[pl.ds(s_idx * chunk, chunk), :],
                dst_ref=out_ref.at[pl.ds(s_idx * chunk, chunk), :],
                send_sem=ag_send_sems.at[t],
                recv_sem=ag_recv_sems.at[t],
                device_id=(right,),
                device_id_type=pl.DeviceIdType.MESH,
            )
            rdma.start()
            rdma.wait()

    return pl.pallas_call(
        body,
        out_shape=jax.ShapeDtypeStruct((m, n), x.dtype),
        in_specs=[pl.BlockSpec(memory_space=pltpu.VMEM)],
        out_specs=pl.BlockSpec(memory_space=pltpu.VMEM),
        scratch_shapes=[
            pltpu.VMEM((N_DEV, chunk, n), x.dtype),
            pltpu.SemaphoreType.DMA((N_DEV,)),
            pltpu.SemaphoreType.DMA((N_DEV,)),
            pltpu.SemaphoreType.DMA((N_DEV - 1,)),
            pltpu.SemaphoreType.DMA((N_DEV - 1,)),
        ],
        compiler_params=pltpu.CompilerParams(collective_id=0),
    )(x)


# device time: 149650 ns/iter; 1.5666x vs baseline; 1.5666x over previous
import jax
import jax.numpy as jnp
from jax import lax
from jax.experimental import pallas as pl
from jax.experimental.pallas import tpu as pltpu

N_DEV = 16
H = N_DEV // 2


def kernel(x):
    m, n = x.shape
    chunk = m // N_DEV

    def body(x_ref, out_ref, ccw_buf, cw_buf,
             ccw_ssem, ccw_rsem, cw_ssem, cw_rsem,
             agcw_ssem, agcw_rsem, agccw_ssem, agccw_rsem):
        d = lax.axis_index("i")
        left = lax.rem(d + N_DEV - 1, N_DEV)
        right = lax.rem(d + 1, N_DEV)

        def cidx(i):
            return lax.rem(i + 2 * N_DEV, N_DEV)

        def xchunk(i):
            return x_ref[pl.ds(cidx(i) * chunk, chunk), :]

        barrier_sem = pltpu.get_barrier_semaphore()
        for nbr in (left, right):
            pl.semaphore_signal(
                barrier_sem, inc=1,
                device_id=(nbr,), device_id_type=pl.DeviceIdType.MESH,
            )
        pl.semaphore_wait(barrier_sem, 2)

        ccw_buf[0, :, :] = xchunk(d - H)
        cw_buf[0, :, :] = xchunk(d + H - 1)
        for k in range(H):
            ccw = pltpu.make_async_remote_copy(
                src_ref=ccw_buf.at[k],
                dst_ref=ccw_buf.at[k + 1],
                send_sem=ccw_ssem.at[k],
                recv_sem=ccw_rsem.at[k + 1],
                device_id=(left,),
                device_id_type=pl.DeviceIdType.MESH,
            )
            ccw.start()
            if k < H - 1:
                cw = pltpu.make_async_remote_copy(
                    src_ref=cw_buf.at[k],
                    dst_ref=cw_buf.at[k + 1],
                    send_sem=cw_ssem.at[k],
                    recv_sem=cw_rsem.at[k + 1],
                    device_id=(right,),
                    device_id_type=pl.DeviceIdType.MESH,
                )
                cw.start()
            ccw.wait()
            ccw_buf[k + 1, :, :] = ccw_buf[k + 1, :, :] + xchunk(d - H + k + 1)
            if k < H - 1:
                cw.wait()
                if k < H - 2:
                    cw_buf[k + 1, :, :] = (
                        cw_buf[k + 1, :, :] + xchunk(d + H - 2 - k)
                    )

        out_ref[pl.ds(d * chunk, chunk), :] = (
            ccw_buf[H, :, :] + cw_buf[H - 1, :, :]
        )

        for t in range(H):
            s_cw = cidx(d - t)
            agcw = pltpu.make_async_remote_copy(
                src_ref=out_ref.at[pl.ds(s_cw * chunk, chunk), :],
                dst_ref=out_ref.at[pl.ds(s_cw * chunk, chunk), :],
                send_sem=agcw_ssem.at[t],
                recv_sem=agcw_rsem.at[t],
                device_id=(right,),
                device_id_type=pl.DeviceIdType.MESH,
            )
            agcw.start()
            if t < H - 1:
                s_ccw = cidx(d + t)
                agccw = pltpu.make_async_remote_copy(
                    src_ref=out_ref.at[pl.ds(s_ccw * chunk, chunk), :],
                    dst_ref=out_ref.at[pl.ds(s_ccw * chunk, chunk), :],
                    send_sem=agccw_ssem.at[t],
                    recv_sem=agccw_rsem.at[t],
                    device_id=(left,),
                    device_id_type=pl.DeviceIdType.MESH,
                )
                agccw.start()
            agcw.wait()
            if t < H - 1:
                agccw.wait()

    return pl.pallas_call(
        body,
        out_shape=jax.ShapeDtypeStruct((m, n), x.dtype),
        in_specs=[pl.BlockSpec(memory_space=pltpu.VMEM)],
        out_specs=pl.BlockSpec(memory_space=pltpu.VMEM),
        scratch_shapes=[
            pltpu.VMEM((H + 1, chunk, n), x.dtype),
            pltpu.VMEM((H, chunk, n), x.dtype),
            pltpu.SemaphoreType.DMA((H,)),
            pltpu.SemaphoreType.DMA((H + 1,)),
            pltpu.SemaphoreType.DMA((H,)),
            pltpu.SemaphoreType.DMA((H,)),
            pltpu.SemaphoreType.DMA((H,)),
            pltpu.SemaphoreType.DMA((H,)),
            pltpu.SemaphoreType.DMA((H,)),
            pltpu.SemaphoreType.DMA((H,)),
        ],
        compiler_params=pltpu.CompilerParams(collective_id=0),
    )(x)


# device time: 109069 ns/iter; 2.1494x vs baseline; 1.3721x over previous
import jax
import jax.numpy as jnp
from jax import lax
from jax.experimental import pallas as pl
from jax.experimental.pallas import tpu as pltpu

N_DEV = 16
H = N_DEV // 2
S = 2


def kernel(x):
    m, n = x.shape
    chunk = m // N_DEV
    sub = chunk // S

    def body(x_ref, out_ref, ccw_buf, cw_buf,
             ccw_ssem, ccw_rsem, cw_ssem, cw_rsem,
             agcw_ssem, agcw_rsem, agccw_ssem, agccw_rsem):
        d = lax.axis_index("i")
        left = lax.rem(d + N_DEV - 1, N_DEV)
        right = lax.rem(d + 1, N_DEV)

        def cidx(i):
            return lax.rem(i + 2 * N_DEV, N_DEV)

        def rs_ccw(k, s):
            return pltpu.make_async_remote_copy(
                src_ref=ccw_buf.at[k, pl.ds(s * sub, sub), :],
                dst_ref=ccw_buf.at[k + 1, pl.ds(s * sub, sub), :],
                send_sem=ccw_ssem.at[k, s],
                recv_sem=ccw_rsem.at[k + 1, s],
                device_id=(left,),
                device_id_type=pl.DeviceIdType.MESH,
            )

        def rs_cw(k, s):
            return pltpu.make_async_remote_copy(
                src_ref=cw_buf.at[k, pl.ds(s * sub, sub), :],
                dst_ref=cw_buf.at[k + 1, pl.ds(s * sub, sub), :],
                send_sem=cw_ssem.at[k, s],
                recv_sem=cw_rsem.at[k + 1, s],
                device_id=(right,),
                device_id_type=pl.DeviceIdType.MESH,
            )

        def ag_cw(t, s):
            c = cidx(d - t)
            return pltpu.make_async_remote_copy(
                src_ref=out_ref.at[pl.ds(c * chunk + s * sub, sub), :],
                dst_ref=out_ref.at[pl.ds(c * chunk + s * sub, sub), :],
                send_sem=agcw_ssem.at[t, s],
                recv_sem=agcw_rsem.at[t, s],
                device_id=(right,),
                device_id_type=pl.DeviceIdType.MESH,
            )

        def ag_ccw(t, s):
            c = cidx(d + t)
            return pltpu.make_async_remote_copy(
                src_ref=out_ref.at[pl.ds(c * chunk + s * sub, sub), :],
                dst_ref=out_ref.at[pl.ds(c * chunk + s * sub, sub), :],
                send_sem=agccw_ssem.at[t, s],
                recv_sem=agccw_rsem.at[t, s],
                device_id=(left,),
                device_id_type=pl.DeviceIdType.MESH,
            )

        def xsub(i, s):
            return x_ref[pl.ds(cidx(i) * chunk + s * sub, sub), :]

        barrier_sem = pltpu.get_barrier_semaphore()
        for nbr in (left, right):
            pl.semaphore_signal(
                barrier_sem, inc=1,
                device_id=(nbr,), device_id_type=pl.DeviceIdType.MESH,
            )
        pl.semaphore_wait(barrier_sem, 2)

        ccw_buf[0, :, :] = x_ref[pl.ds(cidx(d - H) * chunk, chunk), :]
        cw_buf[0, :, :] = x_ref[pl.ds(cidx(d + H - 1) * chunk, chunk), :]
        for s in range(S):
            rs_ccw(0, s).start()
            rs_cw(0, s).start()

        for k in range(H):
            for s in range(S):
                rs_ccw(k, s).wait_recv()
                ccw_buf[k + 1, pl.ds(s * sub, sub), :] = (
                    ccw_buf[k + 1, pl.ds(s * sub, sub), :]
                    + xsub(d - H + k + 1, s)
                )
                if k + 1 < H:
                    rs_ccw(k + 1, s).start()
                else:
                    out_ref[pl.ds(d * chunk + s * sub, sub), :] = (
                        ccw_buf[H, pl.ds(s * sub, sub), :]
                        + cw_buf[H - 1, pl.ds(s * sub, sub), :]
                    )
                    ag_cw(0, s).start()
                    ag_ccw(0, s).start()
            if k < H - 1:
                for s in range(S):
                    rs_cw(k, s).wait_recv()
                    if k < H - 2:
                        cw_buf[k + 1, pl.ds(s * sub, sub), :] = (
                            cw_buf[k + 1, pl.ds(s * sub, sub), :]
                            + xsub(d + H - 2 - k, s)
                        )
                    if k + 1 < H - 1:
                        rs_cw(k + 1, s).start()

        for t in range(H):
            for s in range(S):
                ag_cw(t, s).wait_recv()
                if t + 1 < H:
                    ag_cw(t + 1, s).start()
            if t < H - 1:
                for s in range(S):
                    ag_ccw(t, s).wait_recv()
                    if t + 1 < H - 1:
                        ag_ccw(t + 1, s).start()

        for k in range(H):
            for s in range(S):
                rs_ccw(k, s).wait_send()
                if k < H - 1:
                    rs_cw(k, s).wait_send()
        for t in range(H):
            for s in range(S):
                ag_cw(t, s).wait_send()
                if t < H - 1:
                    ag_ccw(t, s).wait_send()

    return pl.pallas_call(
        body,
        out_shape=jax.ShapeDtypeStruct((m, n), x.dtype),
        in_specs=[pl.BlockSpec(memory_space=pltpu.VMEM)],
        out_specs=pl.BlockSpec(memory_space=pltpu.VMEM),
        scratch_shapes=[
            pltpu.VMEM((H + 1, chunk, n), x.dtype),
            pltpu.VMEM((H, chunk, n), x.dtype),
            pltpu.SemaphoreType.DMA((H, S)),
            pltpu.SemaphoreType.DMA((H + 1, S)),
            pltpu.SemaphoreType.DMA((H, S)),
            pltpu.SemaphoreType.DMA((H, S)),
            pltpu.SemaphoreType.DMA((H, S)),
            pltpu.SemaphoreType.DMA((H, S)),
            pltpu.SemaphoreType.DMA((H, S)),
            pltpu.SemaphoreType.DMA((H, S)),
        ],
        compiler_params=pltpu.CompilerParams(collective_id=0),
    )(x)


# device time: 106517 ns/iter; 2.2009x vs baseline; 1.0240x over previous
import jax
import jax.numpy as jnp
from jax import lax
from jax.experimental import pallas as pl
from jax.experimental.pallas import tpu as pltpu

N_DEV = 16
H = N_DEV // 2
S = 4


def kernel(x):
    m, n = x.shape
    chunk = m // N_DEV
    sub = chunk // S

    def body(x_ref, out_ref, ccw_buf, cw_buf,
             ccw_ssem, ccw_rsem, cw_ssem, cw_rsem,
             agcw_ssem, agcw_rsem, agccw_ssem, agccw_rsem):
        d = lax.axis_index("i")
        left = lax.rem(d + N_DEV - 1, N_DEV)
        right = lax.rem(d + 1, N_DEV)

        def cidx(i):
            return lax.rem(i + 2 * N_DEV, N_DEV)

        def xsub(i, s):
            return x_ref.at[pl.ds(cidx(i) * chunk + s * sub, sub), :]

        def rs_ccw(k, s):
            src = xsub(d - H, s) if k == 0 else ccw_buf.at[k, pl.ds(s * sub, sub), :]
            return pltpu.make_async_remote_copy(
                src_ref=src,
                dst_ref=ccw_buf.at[k + 1, pl.ds(s * sub, sub), :],
                send_sem=ccw_ssem.at[k, s],
                recv_sem=ccw_rsem.at[k + 1, s],
                device_id=(left,),
                device_id_type=pl.DeviceIdType.MESH,
            )

        def rs_cw(k, s):
            src = (
                xsub(d + H - 1, s)
                if k == 0
                else cw_buf.at[k, pl.ds(s * sub, sub), :]
            )
            return pltpu.make_async_remote_copy(
                src_ref=src,
                dst_ref=cw_buf.at[k + 1, pl.ds(s * sub, sub), :],
                send_sem=cw_ssem.at[k, s],
                recv_sem=cw_rsem.at[k + 1, s],
                device_id=(right,),
                device_id_type=pl.DeviceIdType.MESH,
            )

        def ag_cw(t, s):
            c = cidx(d - t)
            return pltpu.make_async_remote_copy(
                src_ref=out_ref.at[pl.ds(c * chunk + s * sub, sub), :],
                dst_ref=out_ref.at[pl.ds(c * chunk + s * sub, sub), :],
                send_sem=agcw_ssem.at[t, s],
                recv_sem=agcw_rsem.at[t, s],
                device_id=(right,),
                device_id_type=pl.DeviceIdType.MESH,
            )

        def ag_ccw(t, s):
            c = cidx(d + t)
            return pltpu.make_async_remote_copy(
                src_ref=out_ref.at[pl.ds(c * chunk + s * sub, sub), :],
                dst_ref=out_ref.at[pl.ds(c * chunk + s * sub, sub), :],
                send_sem=agccw_ssem.at[t, s],
                recv_sem=agccw_rsem.at[t, s],
                device_id=(left,),
                device_id_type=pl.DeviceIdType.MESH,
            )

        barrier_sem = pltpu.get_barrier_semaphore()
        for nbr in (left, right):
            pl.semaphore_signal(
                barrier_sem, inc=1,
                device_id=(nbr,), device_id_type=pl.DeviceIdType.MESH,
            )
        pl.semaphore_wait(barrier_sem, 2)

        for s in range(S):
            rs_ccw(0, s).start()
            rs_cw(0, s).start()

        for k in range(H):
            for s in range(S):
                rs_ccw(k, s).wait_recv()
                ccw_buf[k + 1, pl.ds(s * sub, sub), :] = (
                    ccw_buf[k + 1, pl.ds(s * sub, sub), :]
                    + xsub(d - H + k + 1, s)[:, :]
                )
                if k + 1 < H:
                    rs_ccw(k + 1, s).start()
                else:
                    out_ref[pl.ds(d * chunk + s * sub, sub), :] = (
                        ccw_buf[H, pl.ds(s * sub, sub), :]
                        + cw_buf[H - 1, pl.ds(s * sub, sub), :]
                    )
                    ag_cw(0, s).start()
                    ag_ccw(0, s).start()
                if k < H - 1:
                    rs_cw(k, s).wait_recv()
                    if k < H - 2:
                        cw_buf[k + 1, pl.ds(s * sub, sub), :] = (
                            cw_buf[k + 1, pl.ds(s * sub, sub), :]
                            + xsub(d + H - 2 - k, s)[:, :]
                        )
                    if k + 1 < H - 1:
                        rs_cw(k + 1, s).start()

        for t in range(H):
            for s in range(S):
                ag_cw(t, s).wait_recv()
                if t + 1 < H:
                    ag_cw(t + 1, s).start()
                if t < H - 1:
                    ag_ccw(t, s).wait_recv()
                    if t + 1 < H - 1:
                        ag_ccw(t + 1, s).start()

        for k in range(H):
            for s in range(S):
                rs_ccw(k, s).wait_send()
                if k < H - 1:
                    rs_cw(k, s).wait_send()
        for t in range(H):
            for s in range(S):
                ag_cw(t, s).wait_send()
                if t < H - 1:
                    ag_ccw(t, s).wait_send()

    return pl.pallas_call(
        body,
        out_shape=jax.ShapeDtypeStruct((m, n), x.dtype),
        in_specs=[pl.BlockSpec(memory_space=pltpu.VMEM)],
        out_specs=pl.BlockSpec(memory_space=pltpu.VMEM),
        scratch_shapes=[
            pltpu.VMEM((H + 1, chunk, n), x.dtype),
            pltpu.VMEM((H, chunk, n), x.dtype),
            pltpu.SemaphoreType.DMA((H, S)),
            pltpu.SemaphoreType.DMA((H + 1, S)),
            pltpu.SemaphoreType.DMA((H, S)),
            pltpu.SemaphoreType.DMA((H, S)),
            pltpu.SemaphoreType.DMA((H, S)),
            pltpu.SemaphoreType.DMA((H, S)),
            pltpu.SemaphoreType.DMA((H, S)),
            pltpu.SemaphoreType.DMA((H, S)),
        ],
        compiler_params=pltpu.CompilerParams(collective_id=0),
    )(x)


# device time: 106505 ns/iter; 2.2012x vs baseline; 1.0001x over previous
import jax
import jax.numpy as jnp
from jax import lax
from jax.experimental import pallas as pl
from jax.experimental.pallas import tpu as pltpu

N_DEV = 16
H = N_DEV // 2
S = 4


def kernel(x):
    m, n = x.shape
    chunk = m // N_DEV
    sub = chunk // S

    def body(x_ref, out_ref, ccw_buf, cw_buf,
             ccw_ssem, ccw_rsem, cw_ssem, cw_rsem,
             agcw_ssem, agcw_rsem, agccw_ssem, agccw_rsem):
        d = lax.axis_index("i")
        left = lax.rem(d + N_DEV - 1, N_DEV)
        right = lax.rem(d + 1, N_DEV)

        def cidx(i):
            return lax.rem(i + 2 * N_DEV, N_DEV)

        def xsub(i, s):
            return x_ref.at[pl.ds(cidx(i) * chunk + s * sub, sub), :]

        def rs_ccw(k, s):
            src = xsub(d - H, s) if k == 0 else ccw_buf.at[k, pl.ds(s * sub, sub), :]
            return pltpu.make_async_remote_copy(
                src_ref=src,
                dst_ref=ccw_buf.at[k + 1, pl.ds(s * sub, sub), :],
                send_sem=ccw_ssem.at[k, s],
                recv_sem=ccw_rsem.at[k + 1, s],
                device_id=(left,),
                device_id_type=pl.DeviceIdType.MESH,
            )

        def rs_cw(k, s):
            src = (
                xsub(d + H - 1, s)
                if k == 0
                else cw_buf.at[k, pl.ds(s * sub, sub), :]
            )
            return pltpu.make_async_remote_copy(
                src_ref=src,
                dst_ref=cw_buf.at[k + 1, pl.ds(s * sub, sub), :],
                send_sem=cw_ssem.at[k, s],
                recv_sem=cw_rsem.at[k + 1, s],
                device_id=(right,),
                device_id_type=pl.DeviceIdType.MESH,
            )

        def ag_cw(t, s):
            c = cidx(d - t)
            return pltpu.make_async_remote_copy(
                src_ref=out_ref.at[pl.ds(c * chunk + s * sub, sub), :],
                dst_ref=out_ref.at[pl.ds(c * chunk + s * sub, sub), :],
                send_sem=agcw_ssem.at[t, s],
                recv_sem=agcw_rsem.at[t, s],
                device_id=(right,),
                device_id_type=pl.DeviceIdType.MESH,
            )

        def ag_ccw(t, s):
            c = cidx(d + t)
            return pltpu.make_async_remote_copy(
                src_ref=out_ref.at[pl.ds(c * chunk + s * sub, sub), :],
                dst_ref=out_ref.at[pl.ds(c * chunk + s * sub, sub), :],
                send_sem=agccw_ssem.at[t, s],
                recv_sem=agccw_rsem.at[t, s],
                device_id=(left,),
                device_id_type=pl.DeviceIdType.MESH,
            )

        barrier_sem = pltpu.get_barrier_semaphore()
        for nbr in (left, right):
            pl.semaphore_signal(
                barrier_sem, inc=1,
                device_id=(nbr,), device_id_type=pl.DeviceIdType.MESH,
            )
        pl.semaphore_wait(barrier_sem, 2)

        for s in range(S):
            rs_ccw(0, s).start()
            rs_cw(0, s).start()

        for k in range(H):
            for s in range(S):
                rs_ccw(k, s).wait_recv()
                if k + 1 < H:
                    ccw_buf[k + 1, pl.ds(s * sub, sub), :] = (
                        ccw_buf[k + 1, pl.ds(s * sub, sub), :]
                        + xsub(d - H + k + 1, s)[:, :]
                    )
                    rs_ccw(k + 1, s).start()
                else:
                    out_ref[pl.ds(d * chunk + s * sub, sub), :] = (
                        ccw_buf[H, pl.ds(s * sub, sub), :]
                        + cw_buf[H - 1, pl.ds(s * sub, sub), :]
                        + xsub(d, s)[:, :]
                    )
                    ag_cw(0, s).start()
                    ag_ccw(0, s).start()
                if k < H - 1:
                    rs_cw(k, s).wait_recv()
                    if k < H - 2:
                        cw_buf[k + 1, pl.ds(s * sub, sub), :] = (
                            cw_buf[k + 1, pl.ds(s * sub, sub), :]
                            + xsub(d + H - 2 - k, s)[:, :]
                        )
                    if k + 1 < H - 1:
                        rs_cw(k + 1, s).start()

        for t in range(H):
            for s in range(S):
                ag_cw(t, s).wait_recv()
                if t + 1 < H:
                    ag_cw(t + 1, s).start()
                if t < H - 1:
                    ag_ccw(t, s).wait_recv()
                    if t + 1 < H - 1:
                        ag_ccw(t + 1, s).start()

        for k in range(H):
            for s in range(S):
                rs_ccw(k, s).wait_send()
                if k < H - 1:
                    rs_cw(k, s).wait_send()
        for t in range(H):
            for s in range(S):
                ag_cw(t, s).wait_send()
                if t < H - 1:
                    ag_ccw(t, s).wait_send()

    return pl.pallas_call(
        body,
        out_shape=jax.ShapeDtypeStruct((m, n), x.dtype),
        in_specs=[pl.BlockSpec(memory_space=pltpu.VMEM)],
        out_specs=pl.BlockSpec(memory_space=pltpu.VMEM),
        scratch_shapes=[
            pltpu.VMEM((H + 1, chunk, n), x.dtype),
            pltpu.VMEM((H, chunk, n), x.dtype),
            pltpu.SemaphoreType.DMA((H, S)),
            pltpu.SemaphoreType.DMA((H + 1, S)),
            pltpu.SemaphoreType.DMA((H, S)),
            pltpu.SemaphoreType.DMA((H, S)),
            pltpu.SemaphoreType.DMA((H, S)),
            pltpu.SemaphoreType.DMA((H, S)),
            pltpu.SemaphoreType.DMA((H, S)),
            pltpu.SemaphoreType.DMA((H, S)),
        ],
        compiler_params=pltpu.CompilerParams(collective_id=0),
    )(x)
